# baseline (speedup 1.0000x reference)
import functools

import jax
import jax.numpy as jnp
from jax.experimental import pallas as pl

INPUT_DIM = 768
DICT_SIZES = [3072, 12288, 24576]
MAX_D = 24576
N_TOKENS = 2048
K_STATIC = 32
KB = K_STATIC * N_TOKENS


def _encode_body(x_ref, w_ref, bias_ref, encb_ref, o_ref):
    xc = x_ref[...] - bias_ref[...]
    acc = jax.lax.dot_general(
        xc, w_ref[...], (((1,), (1,)), ((), ())),
        preferred_element_type=jnp.float32)
    o_ref[...] = acc + encb_ref[...]


def _encode(x, enc_w, bias, enc_b):
    M, K = x.shape
    N = enc_w.shape[0]
    BM, BN = 256, 512
    return pl.pallas_call(
        _encode_body,
        grid=(M // BM, N // BN),
        in_specs=[
            pl.BlockSpec((BM, K), lambda i, j: (i, 0)),
            pl.BlockSpec((BN, K), lambda i, j: (j, 0)),
            pl.BlockSpec((1, K), lambda i, j: (0, 0)),
            pl.BlockSpec((1, BN), lambda i, j: (0, j)),
        ],
        out_specs=pl.BlockSpec((BM, BN), lambda i, j: (i, j)),
        out_shape=jax.ShapeDtypeStruct((M, N), jnp.float32),
    )(x, enc_w, bias.reshape(1, K), enc_b.reshape(1, N))


def _decode_body(s_ref, w_ref, bias_ref, o_ref):
    kk = pl.program_id(1)
    acc = jax.lax.dot_general(
        s_ref[...], w_ref[...], (((1,), (0,)), ((), ())),
        preferred_element_type=jnp.float32)

    @pl.when(kk == 0)
    def _():
        o_ref[...] = acc + bias_ref[...]

    @pl.when(kk != 0)
    def _():
        o_ref[...] += acc


def _decode(sparse, w, bias):
    M, D = sparse.shape
    N = w.shape[1]
    BM, BK = 256, 1024
    return pl.pallas_call(
        _decode_body,
        grid=(M // BM, D // BK),
        in_specs=[
            pl.BlockSpec((BM, BK), lambda i, kk: (i, kk)),
            pl.BlockSpec((BK, N), lambda i, kk: (kk, 0)),
            pl.BlockSpec((1, N), lambda i, kk: (0, 0)),
        ],
        out_specs=pl.BlockSpec((BM, N), lambda i, kk: (i, 0)),
        out_shape=jax.ShapeDtypeStruct((M, N), jnp.float32),
    )(sparse, w, bias.reshape(1, N))


def kernel(x, enc_w, enc_b, bias, k):
    b = x.shape[0]
    latents = _encode(x, enc_w, bias, enc_b)
    n_keep = jnp.minimum(k * b, KB)
    recons = []
    for d in DICT_SIZES:
        flat = latents[:, :d].reshape(-1)
        vals, idx = jax.lax.top_k(flat, KB)
        keep = jnp.arange(KB) < n_keep
        vals = jnp.where(keep, vals, jnp.zeros_like(vals))
        sparse = jnp.zeros_like(flat).at[idx].set(vals).reshape(b, d)
        recons.append(_decode(sparse, enc_w[:d, :], bias))
    return tuple(recons)


# speedup vs baseline: 20.4624x; 20.4624x over previous
import functools

import jax
import jax.numpy as jnp
from jax import lax
from jax.experimental import pallas as pl
from jax.experimental.pallas import tpu as pltpu
from jax.experimental.pallas import tpu_sc as plsc

INPUT_DIM = 768
DICT_SIZES = (3072, 12288, 24576)
MAX_D = 24576
N_TOKENS = 2048
K_STATIC = 32
KB = K_STATIC * N_TOKENS

NC = 2
NS = 16
NW = NC * NS
RW = N_TOKENS // NW
CH = 3072
NCH = MAX_D // CH
NVR = CH // 16
GRP = (0, 1, 1, 1, 2, 2, 2, 2)

NB1 = 2048
NB2 = 1024
NB3 = 2048


def _encode_body(x_ref, w_ref, bias_ref, encb_ref, o_ref):
    xc = x_ref[...] - bias_ref[...]
    acc = lax.dot_general(xc, w_ref[...], (((1,), (1,)), ((), ())),
                          preferred_element_type=jnp.float32)
    o_ref[...] = acc + encb_ref[...]


def _encode(x, enc_w, bias, enc_b):
    M, K = x.shape
    N = enc_w.shape[0]
    BM, BN = 256, 512
    return pl.pallas_call(
        _encode_body,
        grid=(M // BM, N // BN),
        in_specs=[
            pl.BlockSpec((BM, K), lambda i, j: (i, 0)),
            pl.BlockSpec((BN, K), lambda i, j: (j, 0)),
            pl.BlockSpec((1, K), lambda i, j: (0, 0)),
            pl.BlockSpec((1, BN), lambda i, j: (0, j)),
        ],
        out_specs=pl.BlockSpec((BM, BN), lambda i, j: (i, j)),
        out_shape=jax.ShapeDtypeStruct((M, N), jnp.float32),
    )(x, enc_w, bias.reshape(1, K), enc_b.reshape(1, N))


def _decode_body(s_ref, w_ref, bias_ref, o_ref):
    kk = pl.program_id(1)
    acc = lax.dot_general(s_ref[...], w_ref[...], (((1,), (0,)), ((), ())),
                          preferred_element_type=jnp.float32)

    @pl.when(kk == 0)
    def _():
        o_ref[...] = acc + bias_ref[...]

    @pl.when(kk != 0)
    def _():
        o_ref[...] += acc


def _decode(sparse, w, bias):
    M, D = sparse.shape
    N = w.shape[1]
    BM, BK = 256, 1024
    return pl.pallas_call(
        _decode_body,
        grid=(M // BM, D // BK),
        in_specs=[
            pl.BlockSpec((BM, BK), lambda i, kk: (i, kk)),
            pl.BlockSpec((BK, N), lambda i, kk: (kk, 0)),
            pl.BlockSpec((1, N), lambda i, kk: (0, 0)),
        ],
        out_specs=pl.BlockSpec((BM, N), lambda i, kk: (i, 0)),
        out_shape=jax.ShapeDtypeStruct((M, N), jnp.float32),
    )(sparse, w, bias.reshape(1, N))


_MESH = plsc.VectorSubcoreMesh(core_axis_name="c", subcore_axis_name="s")


def _lane():
    return lax.iota(jnp.int32, 16)


def _key_of(v):
    u = lax.bitcast_convert_type(v, jnp.uint32)
    m = jnp.where(u >> jnp.uint32(31) > jnp.uint32(0),
                  jnp.uint32(0xFFFFFFFF), jnp.uint32(0x80000000))
    return u ^ m


def _sel(vec, j):
    return jnp.sum(jnp.where(_lane() == j, vec, 0))


def _worker_id():
    return lax.axis_index("s") * NC + lax.axis_index("c")


def _zero_i32(ref, n):
    def z(i, _):
        ref[pl.ds(i * 16, 16)] = jnp.zeros((16,), jnp.int32)
        return 0
    lax.fori_loop(0, n // 16, z, 0)


def _reduce_workers(src_hbm, nper, chunk_ref, acc_ref):
    _zero_i32(acc_ref, nper)

    def wb(w, _):
        pltpu.sync_copy(src_hbm.at[pl.ds(w * nper, nper)], chunk_ref)

        def vb(i, __):
            s = pl.ds(i * 16, 16)
            acc_ref[s] = acc_ref[s] + chunk_ref[s]
            return 0
        lax.fori_loop(0, nper // 16, vb, 0)
        return 0
    lax.fori_loop(0, NW, wb, 0)


def _lane_reduce(hist_ref, red_ref, nbins):
    lane = _lane()

    def rb(o, _):
        base = o * 256
        acc = jnp.zeros((16,), jnp.int32)
        for l in range(16):
            acc = acc + plsc.load_gather(hist_ref, [base + lane * 16 + l])
        red_ref[pl.ds(o * 16, 16)] = acc
        return 0
    lax.fori_loop(0, nbins // 16, rb, 0)


def _find_split(load_cnt, nbins, ca_prev, nk):
    lane = _lane()
    nv = nbins // 16

    def body(i, cy):
        fbin, fca, ca_cum = cy
        vi = nv - 1 - i
        cnt = load_cnt(vi)
        rc = lax.rev(plsc.cumsum(lax.rev(cnt, (0,))), (0,))
        s = rc + ca_cum
        qual = s >= nk
        cand = jnp.max(jnp.where(qual, lane, -1))
        hit = (fbin < 0) & (cand >= 0)
        nb = jnp.where(hit, vi * 16 + cand, fbin)
        nca = jnp.where(hit, _sel(s - cnt, cand), fca)
        return nb, nca, ca_cum + jnp.sum(cnt)

    fbin, fca, _ = lax.fori_loop(
        0, nv, body, (jnp.int32(-1), jnp.int32(0), ca_prev))
    return fbin, fca


def _scan_latents(lat_hbm, chunk_ref, wid, vreg_fn, carry):
    def row_body(rl, cy):
        base = (wid * RW + rl) * MAX_D
        for c in range(NCH):
            pltpu.sync_copy(lat_hbm.at[pl.ds(base + c * CH, CH)], chunk_ref)

            def vb(i, ccy, _c=c, _rl=rl):
                v = chunk_ref[pl.ds(i * 16, 16)]
                return vreg_fn(_c, _rl, i, v, ccy)
            cy = lax.fori_loop(0, NVR, vb, cy)
        return cy
    return lax.fori_loop(0, RW, row_body, carry)


@functools.partial(
    pl.kernel, mesh=_MESH,
    compiler_params=pltpu.CompilerParams(needs_layout_passes=False),
    out_type=jax.ShapeDtypeStruct((NW * 3 * NB1,), jnp.int32),
    scratch_types=[
        pltpu.VMEM((CH,), jnp.float32),
        pltpu.VMEM((3 * NB1 * 16,), jnp.int32),
        pltpu.VMEM((3 * NB1,), jnp.int32),
    ],
)
def _k1_hist1(lat_hbm, hist1_hbm, chunk_v, hist_v, red_v):
    wid = _worker_id()
    _zero_i32(hist_v, 3 * NB1 * 16)
    lane = _lane()
    ones = jnp.ones((16,), jnp.int32)

    def vreg_fn(c, rl, i, v, cy):
        key = _key_of(v)
        b1 = lax.convert_element_type(key >> jnp.uint32(21), jnp.int32)
        base = GRP[c] * (NB1 * 16)
        plsc.addupdate_scatter(hist_v, [base + b1 * 16 + lane], ones)
        return cy

    _scan_latents(lat_hbm, chunk_v, wid, vreg_fn, jnp.int32(0))
    _lane_reduce(hist_v, red_v, 3 * NB1)
    pltpu.sync_copy(red_v, hist1_hbm.at[pl.ds(wid * 3 * NB1, 3 * NB1)])


@functools.partial(
    pl.kernel, mesh=_MESH,
    compiler_params=pltpu.CompilerParams(needs_layout_passes=False),
    out_type=(jax.ShapeDtypeStruct((NW * 3 * NB2,), jnp.int32),
              jax.ShapeDtypeStruct((16,), jnp.int32)),
    scratch_types=[
        pltpu.VMEM((CH,), jnp.float32),
        pltpu.VMEM((3 * NB2 * 16,), jnp.int32),
        pltpu.VMEM((3 * NB1,), jnp.int32),
        pltpu.VMEM((3 * NB1,), jnp.int32),
        pltpu.VMEM((16,), jnp.int32),
    ],
)
def _k2_hist2(lat_hbm, hist1_hbm, nk_hbm, hist2_hbm, thr1_hbm,
              chunk_v, hist_v, red1_v, wchunk_v, thr_v):
    wid = _worker_id()
    _reduce_workers(hist1_hbm, 3 * NB1, wchunk_v, red1_v)
    pltpu.sync_copy(nk_hbm, thr_v)
    nk = jnp.sum(jnp.where(_lane() == 0, thr_v[...], 0))

    B1 = []
    CA1 = []
    for dd in range(3):
        def load_cnt(vi, _dd=dd):
            acc = jnp.zeros((16,), jnp.int32)
            for g in range(_dd + 1):
                acc = acc + red1_v[pl.ds(g * NB1 + vi * 16, 16)]
            return acc
        b, ca = _find_split(load_cnt, NB1, jnp.int32(0), nk)
        B1.append(b)
        CA1.append(ca)

    _zero_i32(hist_v, 3 * NB2 * 16)
    lane = _lane()
    ones = jnp.ones((16,), jnp.int32)

    def vreg_fn(c, rl, i, v, cy):
        key = _key_of(v)
        b1 = lax.convert_element_type(key >> jnp.uint32(21), jnp.int32)
        b2 = lax.convert_element_type(
            (key >> jnp.uint32(11)) & jnp.uint32(NB2 - 1), jnp.int32)
        for dd in range(GRP[c], 3):
            mask = b1 == B1[dd]
            plsc.addupdate_scatter(
                hist_v, [dd * (NB2 * 16) + b2 * 16 + lane], ones, mask=mask)
        return cy

    _scan_latents(lat_hbm, chunk_v, wid, vreg_fn, jnp.int32(0))
    _lane_reduce(hist_v, red1_v, 3 * NB2)
    pltpu.sync_copy(red1_v.at[pl.ds(0, 3 * NB2)],
                    hist2_hbm.at[pl.ds(wid * 3 * NB2, 3 * NB2)])

    thr = jnp.zeros((16,), jnp.int32)
    for dd in range(3):
        thr = jnp.where(lane == dd, B1[dd], thr)
        thr = jnp.where(lane == 3 + dd, CA1[dd], thr)
    red1_v[pl.ds(0, 16)] = thr

    @pl.when(wid == 0)
    def _():
        pltpu.sync_copy(red1_v.at[pl.ds(0, 16)], thr1_hbm)


@functools.partial(
    pl.kernel, mesh=_MESH,
    compiler_params=pltpu.CompilerParams(needs_layout_passes=False),
    out_type=(jax.ShapeDtypeStruct((NW * 3 * NB3,), jnp.int32),
              jax.ShapeDtypeStruct((16,), jnp.int32)),
    scratch_types=[
        pltpu.VMEM((CH,), jnp.float32),
        pltpu.VMEM((3 * NB3 * 16,), jnp.int32),
        pltpu.VMEM((3 * NB2,), jnp.int32),
        pltpu.VMEM((3 * NB2,), jnp.int32),
        pltpu.VMEM((16,), jnp.int32),
    ],
)
def _k3_hist3(lat_hbm, hist2_hbm, thr1_hbm, nk_hbm, hist3_hbm, thr2_hbm,
              chunk_v, hist_v, red2_v, wchunk_v, thr_v):
    wid = _worker_id()
    lane = _lane()
    _reduce_workers(hist2_hbm, 3 * NB2, wchunk_v, red2_v)
    pltpu.sync_copy(thr1_hbm, thr_v)
    t1 = thr_v[...]
    pltpu.sync_copy(nk_hbm, thr_v)
    nk = jnp.sum(jnp.where(lane == 0, thr_v[...], 0))

    T21 = []
    CA2 = []
    for dd in range(3):
        b1 = _sel(t1, dd)
        ca1 = _sel(t1, 3 + dd)

        def load_cnt(vi, _dd=dd):
            return red2_v[pl.ds(_dd * NB2 + vi * 16, 16)]
        b2, ca2 = _find_split(load_cnt, NB2, ca1, nk)
        T21.append(b1 * NB2 + b2)
        CA2.append(ca2)

    _zero_i32(hist_v, 3 * NB3 * 16)
    ones = jnp.ones((16,), jnp.int32)
    T21u = [lax.convert_element_type(t, jnp.uint32) for t in T21]

    def vreg_fn(c, rl, i, v, cy):
        key = _key_of(v)
        p21 = key >> jnp.uint32(11)
        b3 = lax.convert_element_type(key & jnp.uint32(NB3 - 1), jnp.int32)
        for dd in range(GRP[c], 3):
            mask = p21 == T21u[dd]
            plsc.addupdate_scatter(
                hist_v, [dd * (NB3 * 16) + b3 * 16 + lane], ones, mask=mask)
        return cy

    _scan_latents(lat_hbm, chunk_v, wid, vreg_fn, jnp.int32(0))
    _lane_reduce(hist_v, red2_v, 3 * NB3)
    pltpu.sync_copy(red2_v.at[pl.ds(0, 3 * NB3)],
                    hist3_hbm.at[pl.ds(wid * 3 * NB3, 3 * NB3)])

    thr = jnp.zeros((16,), jnp.int32)
    for dd in range(3):
        thr = jnp.where(lane == dd, T21[dd], thr)
        thr = jnp.where(lane == 3 + dd, CA2[dd], thr)
    thr_v[...] = thr

    @pl.when(wid == 0)
    def _():
        pltpu.sync_copy(thr_v, thr2_hbm)


@functools.partial(
    pl.kernel, mesh=_MESH,
    compiler_params=pltpu.CompilerParams(needs_layout_passes=False),
    out_type=(jax.ShapeDtypeStruct((N_TOKENS * DICT_SIZES[0],), jnp.float32),
              jax.ShapeDtypeStruct((N_TOKENS * DICT_SIZES[1],), jnp.float32),
              jax.ShapeDtypeStruct((N_TOKENS * DICT_SIZES[2],), jnp.float32)),
    scratch_types=[
        pltpu.VMEM((CH,), jnp.float32),
        pltpu.VMEM((CH,), jnp.float32),
        pltpu.VMEM((CH,), jnp.float32),
        pltpu.VMEM((CH,), jnp.float32),
        pltpu.VMEM((3 * NB3,), jnp.int32),
        pltpu.VMEM((3 * NB3,), jnp.int32),
        pltpu.VMEM((16,), jnp.int32),
        pltpu.VMEM((32,), jnp.int32),
        pltpu.VMEM((32,), jnp.int32),
        pltpu.SemaphoreType.DMA,
    ],
)
def _k4_select(lat_hbm, hist3_hbm, thr2_hbm, nk_hbm,
               sp1_hbm, sp2_hbm, sp3_hbm,
               chunk_v, ob0_v, ob1_v, ob2_v,
               red3_v, wchunk_v, thr_v, gidx_v, band_v, sem):
    wid = _worker_id()
    lane = _lane()
    _reduce_workers(hist3_hbm, 3 * NB3, wchunk_v, red3_v)
    pltpu.sync_copy(thr2_hbm, thr_v)
    t2 = thr_v[...]
    pltpu.sync_copy(nk_hbm, thr_v)
    nk = jnp.sum(jnp.where(lane == 0, thr_v[...], 0))

    T32 = []
    Q = []
    for dd in range(3):
        t21 = _sel(t2, dd)
        ca2 = _sel(t2, 3 + dd)

        def load_cnt(vi, _dd=dd):
            return red3_v[pl.ds(_dd * NB3 + vi * 16, 16)]
        b3, ca3 = _find_split(load_cnt, NB3, ca2, nk)
        T32.append((lax.convert_element_type(t21, jnp.uint32)
                    << jnp.uint32(11))
                   | lax.convert_element_type(b3, jnp.uint32))
        need = nk - ca3
        off = dd * NB3 + b3
        gidx_v[pl.ds(0, 16)] = lane * (3 * NB3) + off
        gidx_v[pl.ds(16, 16)] = (lane + 16) * (3 * NB3) + off
        pltpu.async_copy(hist3_hbm.at[gidx_v], band_v, sem).wait()
        c0 = band_v[pl.ds(0, 16)]
        c1 = band_v[pl.ds(16, 16)]
        ex0 = plsc.cumsum(c0) - c0
        ex1 = plsc.cumsum(c1) - c1 + jnp.sum(c0)
        pref = jnp.where(wid < 16, _sel(ex0, wid), _sel(ex1, wid - 16))
        cnt = jnp.where(wid < 16, _sel(c0, wid), _sel(c1, wid - 16))
        quota = jnp.clip(need - pref, 0, cnt)
        Q.append(jnp.broadcast_to(quota, (16,)))

    obufs = (ob0_v, ob1_v, ob2_v)
    zero16 = jnp.zeros((16,), jnp.float32)

    def row_body(rl, cy):
        q0, q1, q2 = cy
        qs = [q0, q1, q2]
        r = wid * RW + rl
        base = r * MAX_D
        for c in range(NCH):
            pltpu.sync_copy(lat_hbm.at[pl.ds(base + c * CH, CH)], chunk_v)
            dds = list(range(GRP[c], 3))

            def vb(i, vcy, _dds=tuple(dds)):
                qq = list(vcy)
                v = chunk_v[pl.ds(i * 16, 16)]
                key = _key_of(v)
                for dd in _dds:
                    sure = key > T32[dd]
                    band = key == T32[dd]
                    csum = plsc.cumsum(lax.convert_element_type(
                        band, jnp.int32))
                    fill = band & (csum <= qq[dd])
                    emit = sure | fill
                    obufs[dd][pl.ds(i * 16, 16)] = jnp.where(emit, v, zero16)
                    qq[dd] = qq[dd] - plsc.all_reduce_population_count(fill)
                return tuple(qq)

            qt = lax.fori_loop(0, NVR, vb, tuple(qs))
            qs = list(qt)
            for dd in dds:
                d = DICT_SIZES[dd]
                dst = (sp1_hbm, sp2_hbm, sp3_hbm)[dd]
                pltpu.sync_copy(obufs[dd],
                                dst.at[pl.ds(r * d + c * CH, CH)])
        return tuple(qs)

    lax.fori_loop(0, RW, row_body, tuple(Q))


def kernel(x, enc_w, enc_b, bias, k):
    b = x.shape[0]
    latents = _encode(x, enc_w, bias, enc_b)
    lat_flat = latents.reshape(-1)
    nk = jnp.full((16,), jnp.minimum(k * b, KB), dtype=jnp.int32)

    hist1 = _k1_hist1(lat_flat)
    hist2, thr1 = _k2_hist2(lat_flat, hist1, nk)
    hist3, thr2 = _k3_hist3(lat_flat, hist2, thr1, nk)
    sp1, sp2, sp3 = _k4_select(lat_flat, hist3, thr2, nk)

    recons = []
    for dd, sp in enumerate((sp1, sp2, sp3)):
        d = DICT_SIZES[dd]
        sparse = sp.reshape(b, d)
        recons.append(_decode(sparse, enc_w[:d, :], bias))
    return tuple(recons)


# speedup vs baseline: 25.7302x; 1.2574x over previous
import functools

import jax
import jax.numpy as jnp
from jax import lax
from jax.experimental import pallas as pl
from jax.experimental.pallas import tpu as pltpu
from jax.experimental.pallas import tpu_sc as plsc

INPUT_DIM = 768
DICT_SIZES = (3072, 12288, 24576)
MAX_D = 24576
N_TOKENS = 2048
K_STATIC = 32
KB = K_STATIC * N_TOKENS

NC = 2
NS = 16
NW = NC * NS
RW = N_TOKENS // NW
CH = 3072
NCH = MAX_D // CH
NVR = CH // 16
GRP = (0, 1, 1, 1, 2, 2, 2, 2)

UNROLL = 8

NB1 = 2048
NB2 = 1024
NB3 = 2048


def _encode_body(x_ref, w_ref, bias_ref, encb_ref, o_ref):
    xc = x_ref[...] - bias_ref[...]
    acc = lax.dot_general(xc, w_ref[...], (((1,), (1,)), ((), ())),
                          preferred_element_type=jnp.float32)
    o_ref[...] = acc + encb_ref[...]


def _encode(x, enc_w, bias, enc_b):
    M, K = x.shape
    N = enc_w.shape[0]
    BM, BN = 256, 512
    return pl.pallas_call(
        _encode_body,
        grid=(M // BM, N // BN),
        in_specs=[
            pl.BlockSpec((BM, K), lambda i, j: (i, 0)),
            pl.BlockSpec((BN, K), lambda i, j: (j, 0)),
            pl.BlockSpec((1, K), lambda i, j: (0, 0)),
            pl.BlockSpec((1, BN), lambda i, j: (0, j)),
        ],
        out_specs=pl.BlockSpec((BM, BN), lambda i, j: (i, j)),
        out_shape=jax.ShapeDtypeStruct((M, N), jnp.float32),
    )(x, enc_w, bias.reshape(1, K), enc_b.reshape(1, N))


def _decode_body(s_ref, w_ref, bias_ref, o_ref):
    kk = pl.program_id(1)
    acc = lax.dot_general(s_ref[...], w_ref[...], (((1,), (0,)), ((), ())),
                          preferred_element_type=jnp.float32)

    @pl.when(kk == 0)
    def _():
        o_ref[...] = acc + bias_ref[...]

    @pl.when(kk != 0)
    def _():
        o_ref[...] += acc


def _decode(sparse, w, bias):
    M, D = sparse.shape
    N = w.shape[1]
    BM, BK = 256, 1024
    return pl.pallas_call(
        _decode_body,
        grid=(M // BM, D // BK),
        in_specs=[
            pl.BlockSpec((BM, BK), lambda i, kk: (i, kk)),
            pl.BlockSpec((BK, N), lambda i, kk: (kk, 0)),
            pl.BlockSpec((1, N), lambda i, kk: (0, 0)),
        ],
        out_specs=pl.BlockSpec((BM, N), lambda i, kk: (i, 0)),
        out_shape=jax.ShapeDtypeStruct((M, N), jnp.float32),
    )(sparse, w, bias.reshape(1, N))


_MESH = plsc.VectorSubcoreMesh(core_axis_name="c", subcore_axis_name="s")


def _lane():
    return lax.iota(jnp.int32, 16)


def _key_of(v):
    u = lax.bitcast_convert_type(v, jnp.uint32)
    m = jnp.where(u >> jnp.uint32(31) > jnp.uint32(0),
                  jnp.uint32(0xFFFFFFFF), jnp.uint32(0x80000000))
    return u ^ m


def _sel(vec, j):
    return jnp.sum(jnp.where(_lane() == j, vec, 0))


def _worker_id():
    return lax.axis_index("s") * NC + lax.axis_index("c")


def _zero_i32(ref, n):
    def z(i, _):
        ref[pl.ds(i * 16, 16)] = jnp.zeros((16,), jnp.int32)
        return 0
    lax.fori_loop(0, n // 16, z, 0)


def _reduce_workers(src_hbm, nper, chunk_ref, acc_ref):
    _zero_i32(acc_ref, nper)

    def wb(w, _):
        pltpu.sync_copy(src_hbm.at[pl.ds(w * nper, nper)], chunk_ref)

        def vb(i, __):
            s = pl.ds(i * 16, 16)
            acc_ref[s] = acc_ref[s] + chunk_ref[s]
            return 0
        lax.fori_loop(0, nper // 16, vb, 0)
        return 0
    lax.fori_loop(0, NW, wb, 0)


def _lane_reduce(hist_ref, red_ref, nbins):
    lane = _lane()

    def rb(o, _):
        base = o * 256
        acc = jnp.zeros((16,), jnp.int32)
        for l in range(16):
            acc = acc + plsc.load_gather(hist_ref, [base + lane * 16 + l])
        red_ref[pl.ds(o * 16, 16)] = acc
        return 0
    lax.fori_loop(0, nbins // 16, rb, 0)


def _find_split(load_cnt, nbins, ca_prev, nk):
    lane = _lane()
    nv = nbins // 16

    def body(i, cy):
        fbin, fca, ca_cum = cy
        vi = nv - 1 - i
        cnt = load_cnt(vi)
        rc = lax.rev(plsc.cumsum(lax.rev(cnt, (0,))), (0,))
        s = rc + ca_cum
        qual = s >= nk
        cand = jnp.max(jnp.where(qual, lane, -1))
        hit = (fbin < 0) & (cand >= 0)
        nb = jnp.where(hit, vi * 16 + cand, fbin)
        nca = jnp.where(hit, _sel(s - cnt, cand), fca)
        return nb, nca, ca_cum + jnp.sum(cnt)

    fbin, fca, _ = lax.fori_loop(
        0, nv, body, (jnp.int32(-1), jnp.int32(0), ca_prev))
    return fbin, fca


def _scan_latents(lat_hbm, bufs, sems, wid, vreg_fn, carry):
    def slice_of(rl, c):
        return lat_hbm.at[pl.ds((wid * RW + rl) * MAX_D + c * CH, CH)]

    def start(rl, c, p):
        pltpu.make_async_copy(slice_of(rl, c), bufs[p], sems[p]).start()

    def wait(rl, c, p):
        pltpu.make_async_copy(slice_of(rl, c), bufs[p], sems[p]).wait()

    start(0, 0, 0)

    def row_body(rl, cy):
        for c in range(NCH):
            p = c % 2
            wait(rl, c, p)
            if c < NCH - 1:
                start(rl, c + 1, (c + 1) % 2)
            else:
                @pl.when(rl + 1 < RW)
                def _():
                    start(rl + 1, 0, 0)

            def vb(i, ccy, _c=c, _rl=rl, _p=p):
                for u in range(UNROLL):
                    v = bufs[_p][pl.ds((i * UNROLL + u) * 16, 16)]
                    ccy = vreg_fn(_c, _rl, i * UNROLL + u, v, ccy)
                return ccy
            cy = lax.fori_loop(0, NVR // UNROLL, vb, cy)
        return cy
    return lax.fori_loop(0, RW, row_body, carry)


@functools.partial(
    pl.kernel, mesh=_MESH,
    compiler_params=pltpu.CompilerParams(needs_layout_passes=False),
    out_type=jax.ShapeDtypeStruct((NW * 3 * NB1,), jnp.int32),
    scratch_types=[
        pltpu.VMEM((CH,), jnp.float32),
        pltpu.VMEM((CH,), jnp.float32),
        pltpu.VMEM((3 * NB1 * 16,), jnp.int32),
        pltpu.VMEM((3 * NB1,), jnp.int32),
        pltpu.SemaphoreType.DMA,
        pltpu.SemaphoreType.DMA,
    ],
)
def _k1_hist1(lat_hbm, hist1_hbm, chunk0_v, chunk1_v, hist_v, red_v,
              sem0, sem1):
    wid = _worker_id()
    _zero_i32(hist_v, 3 * NB1 * 16)
    lane = _lane()
    ones = jnp.ones((16,), jnp.int32)

    def vreg_fn(c, rl, i, v, cy):
        key = _key_of(v)
        b1 = lax.convert_element_type(key >> jnp.uint32(21), jnp.int32)
        base = GRP[c] * (NB1 * 16)
        plsc.addupdate_scatter(hist_v, [base + b1 * 16 + lane], ones)
        return cy

    _scan_latents(lat_hbm, (chunk0_v, chunk1_v), (sem0, sem1),
                  wid, vreg_fn, jnp.int32(0))
    _lane_reduce(hist_v, red_v, 3 * NB1)
    pltpu.sync_copy(red_v, hist1_hbm.at[pl.ds(wid * 3 * NB1, 3 * NB1)])


@functools.partial(
    pl.kernel, mesh=_MESH,
    compiler_params=pltpu.CompilerParams(needs_layout_passes=False),
    out_type=(jax.ShapeDtypeStruct((NW * 3 * NB2,), jnp.int32),
              jax.ShapeDtypeStruct((16,), jnp.int32)),
    scratch_types=[
        pltpu.VMEM((CH,), jnp.float32),
        pltpu.VMEM((CH,), jnp.float32),
        pltpu.VMEM((3 * NB2 * 16,), jnp.int32),
        pltpu.VMEM((3 * NB1,), jnp.int32),
        pltpu.VMEM((3 * NB1,), jnp.int32),
        pltpu.VMEM((16,), jnp.int32),
        pltpu.SemaphoreType.DMA,
        pltpu.SemaphoreType.DMA,
    ],
)
def _k2_hist2(lat_hbm, hist1_hbm, nk_hbm, hist2_hbm, thr1_hbm,
              chunk0_v, chunk1_v, hist_v, red1_v, wchunk_v, thr_v,
              sem0, sem1):
    wid = _worker_id()
    _reduce_workers(hist1_hbm, 3 * NB1, wchunk_v, red1_v)
    pltpu.sync_copy(nk_hbm, thr_v)
    nk = jnp.sum(jnp.where(_lane() == 0, thr_v[...], 0))

    B1 = []
    CA1 = []
    for dd in range(3):
        def load_cnt(vi, _dd=dd):
            acc = jnp.zeros((16,), jnp.int32)
            for g in range(_dd + 1):
                acc = acc + red1_v[pl.ds(g * NB1 + vi * 16, 16)]
            return acc
        b, ca = _find_split(load_cnt, NB1, jnp.int32(0), nk)
        B1.append(b)
        CA1.append(ca)

    _zero_i32(hist_v, 3 * NB2 * 16)
    lane = _lane()
    ones = jnp.ones((16,), jnp.int32)

    def vreg_fn(c, rl, i, v, cy):
        key = _key_of(v)
        b1 = lax.convert_element_type(key >> jnp.uint32(21), jnp.int32)
        b2 = lax.convert_element_type(
            (key >> jnp.uint32(11)) & jnp.uint32(NB2 - 1), jnp.int32)
        for dd in range(GRP[c], 3):
            mask = b1 == B1[dd]
            plsc.addupdate_scatter(
                hist_v, [dd * (NB2 * 16) + b2 * 16 + lane], ones, mask=mask)
        return cy

    _scan_latents(lat_hbm, (chunk0_v, chunk1_v), (sem0, sem1),
                  wid, vreg_fn, jnp.int32(0))
    _lane_reduce(hist_v, red1_v, 3 * NB2)
    pltpu.sync_copy(red1_v.at[pl.ds(0, 3 * NB2)],
                    hist2_hbm.at[pl.ds(wid * 3 * NB2, 3 * NB2)])

    thr = jnp.zeros((16,), jnp.int32)
    for dd in range(3):
        thr = jnp.where(lane == dd, B1[dd], thr)
        thr = jnp.where(lane == 3 + dd, CA1[dd], thr)
    red1_v[pl.ds(0, 16)] = thr

    @pl.when(wid == 0)
    def _():
        pltpu.sync_copy(red1_v.at[pl.ds(0, 16)], thr1_hbm)


@functools.partial(
    pl.kernel, mesh=_MESH,
    compiler_params=pltpu.CompilerParams(needs_layout_passes=False),
    out_type=(jax.ShapeDtypeStruct((NW * 3 * NB3,), jnp.int32),
              jax.ShapeDtypeStruct((16,), jnp.int32)),
    scratch_types=[
        pltpu.VMEM((CH,), jnp.float32),
        pltpu.VMEM((CH,), jnp.float32),
        pltpu.VMEM((3 * NB3 * 16,), jnp.int32),
        pltpu.VMEM((3 * NB2,), jnp.int32),
        pltpu.VMEM((3 * NB2,), jnp.int32),
        pltpu.VMEM((16,), jnp.int32),
        pltpu.SemaphoreType.DMA,
        pltpu.SemaphoreType.DMA,
    ],
)
def _k3_hist3(lat_hbm, hist2_hbm, thr1_hbm, nk_hbm, hist3_hbm, thr2_hbm,
              chunk0_v, chunk1_v, hist_v, red2_v, wchunk_v, thr_v,
              sem0, sem1):
    wid = _worker_id()
    lane = _lane()
    _reduce_workers(hist2_hbm, 3 * NB2, wchunk_v, red2_v)
    pltpu.sync_copy(thr1_hbm, thr_v)
    t1 = thr_v[...]
    pltpu.sync_copy(nk_hbm, thr_v)
    nk = jnp.sum(jnp.where(lane == 0, thr_v[...], 0))

    T21 = []
    CA2 = []
    for dd in range(3):
        b1 = _sel(t1, dd)
        ca1 = _sel(t1, 3 + dd)

        def load_cnt(vi, _dd=dd):
            return red2_v[pl.ds(_dd * NB2 + vi * 16, 16)]
        b2, ca2 = _find_split(load_cnt, NB2, ca1, nk)
        T21.append(b1 * NB2 + b2)
        CA2.append(ca2)

    _zero_i32(hist_v, 3 * NB3 * 16)
    ones = jnp.ones((16,), jnp.int32)
    T21u = [lax.convert_element_type(t, jnp.uint32) for t in T21]

    def vreg_fn(c, rl, i, v, cy):
        key = _key_of(v)
        p21 = key >> jnp.uint32(11)
        b3 = lax.convert_element_type(key & jnp.uint32(NB3 - 1), jnp.int32)
        for dd in range(GRP[c], 3):
            mask = p21 == T21u[dd]
            plsc.addupdate_scatter(
                hist_v, [dd * (NB3 * 16) + b3 * 16 + lane], ones, mask=mask)
        return cy

    _scan_latents(lat_hbm, (chunk0_v, chunk1_v), (sem0, sem1),
                  wid, vreg_fn, jnp.int32(0))
    _lane_reduce(hist_v, red2_v, 3 * NB3)
    pltpu.sync_copy(red2_v.at[pl.ds(0, 3 * NB3)],
                    hist3_hbm.at[pl.ds(wid * 3 * NB3, 3 * NB3)])

    thr = jnp.zeros((16,), jnp.int32)
    for dd in range(3):
        thr = jnp.where(lane == dd, T21[dd], thr)
        thr = jnp.where(lane == 3 + dd, CA2[dd], thr)
    thr_v[...] = thr

    @pl.when(wid == 0)
    def _():
        pltpu.sync_copy(thr_v, thr2_hbm)


@functools.partial(
    pl.kernel, mesh=_MESH,
    compiler_params=pltpu.CompilerParams(needs_layout_passes=False),
    out_type=(jax.ShapeDtypeStruct((N_TOKENS * DICT_SIZES[0],), jnp.float32),
              jax.ShapeDtypeStruct((N_TOKENS * DICT_SIZES[1],), jnp.float32),
              jax.ShapeDtypeStruct((N_TOKENS * DICT_SIZES[2],), jnp.float32)),
    scratch_types=(
        [pltpu.VMEM((CH,), jnp.float32) for _ in range(7)]
        + [
            pltpu.VMEM((3 * NB3,), jnp.int32),
            pltpu.VMEM((3 * NB3,), jnp.int32),
            pltpu.VMEM((16,), jnp.int32),
            pltpu.VMEM((32,), jnp.int32),
            pltpu.VMEM((32,), jnp.int32),
        ]
        + [pltpu.SemaphoreType.DMA for _ in range(8)]
    ),
)
def _k4_select(lat_hbm, hist3_hbm, thr2_hbm, nk_hbm,
               sp1_hbm, sp2_hbm, sp3_hbm,
               chunk0_v, chunk1_v, ob00_v, ob10_v, ob11_v, ob20_v, ob21_v,
               red3_v, wchunk_v, thr_v, gidx_v, band_v,
               sem, semi0, semi1, so00, so10, so11, so20, so21):
    wid = _worker_id()
    lane = _lane()
    _reduce_workers(hist3_hbm, 3 * NB3, wchunk_v, red3_v)
    pltpu.sync_copy(thr2_hbm, thr_v)
    t2 = thr_v[...]
    pltpu.sync_copy(nk_hbm, thr_v)
    nk = jnp.sum(jnp.where(lane == 0, thr_v[...], 0))

    T32 = []
    Q = []
    for dd in range(3):
        t21 = _sel(t2, dd)
        ca2 = _sel(t2, 3 + dd)

        def load_cnt(vi, _dd=dd):
            return red3_v[pl.ds(_dd * NB3 + vi * 16, 16)]
        b3, ca3 = _find_split(load_cnt, NB3, ca2, nk)
        T32.append((lax.convert_element_type(t21, jnp.uint32)
                    << jnp.uint32(11))
                   | lax.convert_element_type(b3, jnp.uint32))
        need = nk - ca3
        off = dd * NB3 + b3
        gidx_v[pl.ds(0, 16)] = lane * (3 * NB3) + off
        gidx_v[pl.ds(16, 16)] = (lane + 16) * (3 * NB3) + off
        pltpu.async_copy(hist3_hbm.at[gidx_v], band_v, sem).wait()
        c0 = band_v[pl.ds(0, 16)]
        c1 = band_v[pl.ds(16, 16)]
        ex0 = plsc.cumsum(c0) - c0
        ex1 = plsc.cumsum(c1) - c1 + jnp.sum(c0)
        pref = jnp.where(wid < 16, _sel(ex0, wid), _sel(ex1, wid - 16))
        cnt = jnp.where(wid < 16, _sel(c0, wid), _sel(c1, wid - 16))
        quota = jnp.clip(need - pref, 0, cnt)
        Q.append(jnp.broadcast_to(quota, (16,)))

    OB = {0: {0: ob00_v}, 1: {0: ob10_v, 1: ob11_v}, 2: {0: ob20_v, 1: ob21_v}}
    SO = {0: {0: so00}, 1: {0: so10, 1: so11}, 2: {0: so20, 1: so21}}
    sps = (sp1_hbm, sp2_hbm, sp3_hbm)
    inbufs = (chunk0_v, chunk1_v)
    insems = (semi0, semi1)
    zero16 = jnp.zeros((16,), jnp.float32)

    def in_slice(rl, c):
        return lat_hbm.at[pl.ds((wid * RW + rl) * MAX_D + c * CH, CH)]

    def out_dma(dd, p, r, c):
        d = DICT_SIZES[dd]
        return pltpu.make_async_copy(
            OB[dd][p], sps[dd].at[pl.ds(r * d + c * CH, CH)], SO[dd][p])

    pltpu.make_async_copy(in_slice(0, 0), inbufs[0], insems[0]).start()

    def row_body(rl, cy):
        qs = list(cy)
        r = wid * RW + rl
        for c in range(NCH):
            p = c % 2
            pltpu.make_async_copy(in_slice(rl, c), inbufs[p], insems[p]).wait()
            if c < NCH - 1:
                pltpu.make_async_copy(
                    in_slice(rl, c + 1), inbufs[(c + 1) % 2],
                    insems[(c + 1) % 2]).start()
            else:
                @pl.when(rl + 1 < RW)
                def _():
                    pltpu.make_async_copy(
                        in_slice(rl + 1, 0), inbufs[0], insems[0]).start()
            dds = list(range(GRP[c], 3))
            for dd in dds:
                if c <= 1:
                    @pl.when(rl > 0)
                    def _(dd=dd):
                        out_dma(dd, p, r, c).wait()
                else:
                    out_dma(dd, p, r, c).wait()

            def vb(i, vcy, _dds=tuple(dds), _p=p):
                qq = list(vcy)
                for u in range(UNROLL):
                    iv = i * UNROLL + u
                    v = inbufs[_p][pl.ds(iv * 16, 16)]
                    key = _key_of(v)
                    for dd in _dds:
                        sure = key > T32[dd]
                        band = key == T32[dd]
                        csum = plsc.cumsum(lax.convert_element_type(
                            band, jnp.int32))
                        fill = band & (csum <= qq[dd])
                        emit = sure | fill
                        OB[dd][_p][pl.ds(iv * 16, 16)] = jnp.where(
                            emit, v, zero16)
                        qq[dd] = qq[dd] - plsc.all_reduce_population_count(
                            fill)
                return tuple(qq)

            qt = lax.fori_loop(0, NVR // UNROLL, vb, tuple(qs))
            qs = list(qt)
            for dd in dds:
                out_dma(dd, p, r, c).start()
        return tuple(qs)

    lax.fori_loop(0, RW, row_body, tuple(Q))
    last_r = N_TOKENS - 1
    for dd, p, c_last in ((0, 0, 0), (1, 0, 2), (1, 1, 3), (2, 0, 6), (2, 1, 7)):
        out_dma(dd, p, last_r, c_last).wait()


def kernel(x, enc_w, enc_b, bias, k):
    b = x.shape[0]
    latents = _encode(x, enc_w, bias, enc_b)
    lat_flat = latents.reshape(-1)
    nk = jnp.full((16,), jnp.minimum(k * b, KB), dtype=jnp.int32)

    hist1 = _k1_hist1(lat_flat)
    hist2, thr1 = _k2_hist2(lat_flat, hist1, nk)
    hist3, thr2 = _k3_hist3(lat_flat, hist2, thr1, nk)
    sp1, sp2, sp3 = _k4_select(lat_flat, hist3, thr2, nk)

    recons = []
    for dd, sp in enumerate((sp1, sp2, sp3)):
        d = DICT_SIZES[dd]
        sparse = sp.reshape(b, d)
        recons.append(_decode(sparse, enc_w[:d, :], bias))
    return tuple(recons)


# speedup vs baseline: 37.7574x; 1.4674x over previous
import functools

import jax
import jax.numpy as jnp
from jax import lax
from jax.experimental import pallas as pl
from jax.experimental.pallas import tpu as pltpu
from jax.experimental.pallas import tpu_sc as plsc

INPUT_DIM = 768
DICT_SIZES = (3072, 12288, 24576)
MAX_D = 24576
N_TOKENS = 2048
K_STATIC = 32
KB = K_STATIC * N_TOKENS

NC = 2
NS = 16
NW = NC * NS
RW = N_TOKENS // NW
CH = 3072
NCH = MAX_D // CH
NVR = CH // 16
GRP = (0, 1, 1, 1, 2, 2, 2, 2)

UNROLL = 8

NB1 = 2048
NB2 = 1024
NB3 = 2048


def _encode_body(x_ref, w_ref, bias_ref, encb_ref, o_ref):
    xc = x_ref[...] - bias_ref[...]
    acc = lax.dot_general(xc, w_ref[...], (((1,), (1,)), ((), ())),
                          preferred_element_type=jnp.float32)
    o_ref[...] = acc + encb_ref[...]


def _encode(x, enc_w, bias, enc_b):
    M, K = x.shape
    N = enc_w.shape[0]
    BM, BN = 256, 512
    return pl.pallas_call(
        _encode_body,
        grid=(M // BM, N // BN),
        in_specs=[
            pl.BlockSpec((BM, K), lambda i, j: (i, 0)),
            pl.BlockSpec((BN, K), lambda i, j: (j, 0)),
            pl.BlockSpec((1, K), lambda i, j: (0, 0)),
            pl.BlockSpec((1, BN), lambda i, j: (0, j)),
        ],
        out_specs=pl.BlockSpec((BM, BN), lambda i, j: (i, j)),
        out_shape=jax.ShapeDtypeStruct((M, N), jnp.float32),
    )(x, enc_w, bias.reshape(1, K), enc_b.reshape(1, N))


def _decode_body(s_ref, w_ref, bias_ref, o_ref):
    kk = pl.program_id(1)
    acc = lax.dot_general(s_ref[...], w_ref[...], (((1,), (0,)), ((), ())),
                          preferred_element_type=jnp.float32)

    @pl.when(kk == 0)
    def _():
        o_ref[...] = acc + bias_ref[...]

    @pl.when(kk != 0)
    def _():
        o_ref[...] += acc


def _decode(sparse, w, bias):
    M, D = sparse.shape
    N = w.shape[1]
    BM, BK = 256, 1024
    return pl.pallas_call(
        _decode_body,
        grid=(M // BM, D // BK),
        in_specs=[
            pl.BlockSpec((BM, BK), lambda i, kk: (i, kk)),
            pl.BlockSpec((BK, N), lambda i, kk: (kk, 0)),
            pl.BlockSpec((1, N), lambda i, kk: (0, 0)),
        ],
        out_specs=pl.BlockSpec((BM, N), lambda i, kk: (i, 0)),
        out_shape=jax.ShapeDtypeStruct((M, N), jnp.float32),
    )(sparse, w, bias.reshape(1, N))


_MESH = plsc.VectorSubcoreMesh(core_axis_name="c", subcore_axis_name="s")


def _lane():
    return lax.iota(jnp.int32, 16)


def _key_of(v):
    u = lax.bitcast_convert_type(v, jnp.uint32)
    m = jnp.where(u >> jnp.uint32(31) > jnp.uint32(0),
                  jnp.uint32(0xFFFFFFFF), jnp.uint32(0x80000000))
    return u ^ m


def _sel(vec, j):
    return jnp.sum(jnp.where(_lane() == j, vec, 0))


def _worker_id():
    return lax.axis_index("s") * NC + lax.axis_index("c")


def _zero_i32(ref, n):
    def z(i, _):
        ref[pl.ds(i * 16, 16)] = jnp.zeros((16,), jnp.int32)
        return 0
    lax.fori_loop(0, n // 16, z, 0)


def _reduce_workers(src_hbm, nper, chunk_ref, acc_ref):
    _zero_i32(acc_ref, nper)

    def wb(w, _):
        pltpu.sync_copy(src_hbm.at[pl.ds(w * nper, nper)], chunk_ref)

        def vb(i, __):
            s = pl.ds(i * 16, 16)
            acc_ref[s] = acc_ref[s] + chunk_ref[s]
            return 0
        lax.fori_loop(0, nper // 16, vb, 0)
        return 0
    lax.fori_loop(0, NW, wb, 0)


def _lane_reduce(hist_ref, red_ref, nbins):
    lane = _lane()

    def rb(o, _):
        base = o * 256
        acc = jnp.zeros((16,), jnp.int32)
        for l in range(16):
            acc = acc + plsc.load_gather(hist_ref, [base + lane * 16 + l])
        red_ref[pl.ds(o * 16, 16)] = acc
        return 0
    lax.fori_loop(0, nbins // 16, rb, 0)


def _find_split(load_cnt, nbins, ca_prev, nk):
    lane = _lane()
    nv = nbins // 16

    def body(i, cy):
        fbin, fca, ca_cum = cy
        vi = nv - 1 - i
        cnt = load_cnt(vi)
        rc = lax.rev(plsc.cumsum(lax.rev(cnt, (0,))), (0,))
        s = rc + ca_cum
        qual = s >= nk
        cand = jnp.max(jnp.where(qual, lane, -1))
        hit = (fbin < 0) & (cand >= 0)
        nb = jnp.where(hit, vi * 16 + cand, fbin)
        nca = jnp.where(hit, _sel(s - cnt, cand), fca)
        return nb, nca, ca_cum + jnp.sum(cnt)

    fbin, fca, _ = lax.fori_loop(
        0, nv, body, (jnp.int32(-1), jnp.int32(0), ca_prev))
    return fbin, fca


def _scan_latents(lat_hbm, bufs, sems, wid, vreg_fn, carry):
    def slice_of(rl, c):
        return lat_hbm.at[pl.ds((wid * RW + rl) * MAX_D + c * CH, CH)]

    def start(rl, c, p):
        pltpu.make_async_copy(slice_of(rl, c), bufs[p], sems[p]).start()

    def wait(rl, c, p):
        pltpu.make_async_copy(slice_of(rl, c), bufs[p], sems[p]).wait()

    start(0, 0, 0)

    def row_body(rl, cy):
        for c in range(NCH):
            p = c % 2
            wait(rl, c, p)
            if c < NCH - 1:
                start(rl, c + 1, (c + 1) % 2)
            else:
                @pl.when(rl + 1 < RW)
                def _():
                    start(rl + 1, 0, 0)

            @plsc.parallel_loop(0, NVR, step=1, unroll=UNROLL, carry=cy)
            def _body(i, ccy, _c=c, _rl=rl, _p=p):
                v = bufs[_p][pl.ds(i * 16, 16)]
                return vreg_fn(_c, _rl, i, v, ccy)
            cy = _body
        return cy
    return lax.fori_loop(0, RW, row_body, carry)


@functools.partial(
    pl.kernel, mesh=_MESH,
    compiler_params=pltpu.CompilerParams(needs_layout_passes=False),
    out_type=jax.ShapeDtypeStruct((NW * 3 * NB1,), jnp.int32),
    scratch_types=[
        pltpu.VMEM((CH,), jnp.float32),
        pltpu.VMEM((CH,), jnp.float32),
        pltpu.VMEM((3 * NB1 * 16,), jnp.int32),
        pltpu.VMEM((3 * NB1,), jnp.int32),
        pltpu.SemaphoreType.DMA,
        pltpu.SemaphoreType.DMA,
    ],
)
def _k1_hist1(lat_hbm, hist1_hbm, chunk0_v, chunk1_v, hist_v, red_v,
              sem0, sem1):
    wid = _worker_id()
    _zero_i32(hist_v, 3 * NB1 * 16)
    lane = _lane()
    ones = jnp.ones((16,), jnp.int32)

    def vreg_fn(c, rl, i, v, cy):
        key = _key_of(v)
        b1 = lax.convert_element_type(key >> jnp.uint32(21), jnp.int32)
        base = GRP[c] * (NB1 * 16)
        plsc.addupdate_scatter(hist_v, [base + b1 * 16 + lane], ones)
        return cy

    _scan_latents(lat_hbm, (chunk0_v, chunk1_v), (sem0, sem1),
                  wid, vreg_fn, jnp.int32(0))
    _lane_reduce(hist_v, red_v, 3 * NB1)
    pltpu.sync_copy(red_v, hist1_hbm.at[pl.ds(wid * 3 * NB1, 3 * NB1)])


@functools.partial(
    pl.kernel, mesh=_MESH,
    compiler_params=pltpu.CompilerParams(needs_layout_passes=False),
    out_type=(jax.ShapeDtypeStruct((NW * 3 * NB2,), jnp.int32),
              jax.ShapeDtypeStruct((16,), jnp.int32)),
    scratch_types=[
        pltpu.VMEM((CH,), jnp.float32),
        pltpu.VMEM((CH,), jnp.float32),
        pltpu.VMEM((3 * NB2 * 16,), jnp.int32),
        pltpu.VMEM((3 * NB1,), jnp.int32),
        pltpu.VMEM((3 * NB1,), jnp.int32),
        pltpu.VMEM((16,), jnp.int32),
        pltpu.SemaphoreType.DMA,
        pltpu.SemaphoreType.DMA,
    ],
)
def _k2_hist2(lat_hbm, hist1_hbm, nk_hbm, hist2_hbm, thr1_hbm,
              chunk0_v, chunk1_v, hist_v, red1_v, wchunk_v, thr_v,
              sem0, sem1):
    wid = _worker_id()
    _reduce_workers(hist1_hbm, 3 * NB1, wchunk_v, red1_v)
    pltpu.sync_copy(nk_hbm, thr_v)
    nk = jnp.sum(jnp.where(_lane() == 0, thr_v[...], 0))

    B1 = []
    CA1 = []
    for dd in range(3):
        def load_cnt(vi, _dd=dd):
            acc = jnp.zeros((16,), jnp.int32)
            for g in range(_dd + 1):
                acc = acc + red1_v[pl.ds(g * NB1 + vi * 16, 16)]
            return acc
        b, ca = _find_split(load_cnt, NB1, jnp.int32(0), nk)
        B1.append(b)
        CA1.append(ca)

    _zero_i32(hist_v, 3 * NB2 * 16)
    lane = _lane()
    ones = jnp.ones((16,), jnp.int32)

    def vreg_fn(c, rl, i, v, cy):
        key = _key_of(v)
        b1 = lax.convert_element_type(key >> jnp.uint32(21), jnp.int32)
        b2 = lax.convert_element_type(
            (key >> jnp.uint32(11)) & jnp.uint32(NB2 - 1), jnp.int32)
        for dd in range(GRP[c], 3):
            mask = b1 == B1[dd]
            plsc.addupdate_scatter(
                hist_v, [dd * (NB2 * 16) + b2 * 16 + lane], ones, mask=mask)
        return cy

    _scan_latents(lat_hbm, (chunk0_v, chunk1_v), (sem0, sem1),
                  wid, vreg_fn, jnp.int32(0))
    _lane_reduce(hist_v, red1_v, 3 * NB2)
    pltpu.sync_copy(red1_v.at[pl.ds(0, 3 * NB2)],
                    hist2_hbm.at[pl.ds(wid * 3 * NB2, 3 * NB2)])

    thr = jnp.zeros((16,), jnp.int32)
    for dd in range(3):
        thr = jnp.where(lane == dd, B1[dd], thr)
        thr = jnp.where(lane == 3 + dd, CA1[dd], thr)
    red1_v[pl.ds(0, 16)] = thr

    @pl.when(wid == 0)
    def _():
        pltpu.sync_copy(red1_v.at[pl.ds(0, 16)], thr1_hbm)


@functools.partial(
    pl.kernel, mesh=_MESH,
    compiler_params=pltpu.CompilerParams(needs_layout_passes=False),
    out_type=(jax.ShapeDtypeStruct((NW * 3 * NB3,), jnp.int32),
              jax.ShapeDtypeStruct((16,), jnp.int32)),
    scratch_types=[
        pltpu.VMEM((CH,), jnp.float32),
        pltpu.VMEM((CH,), jnp.float32),
        pltpu.VMEM((3 * NB3 * 16,), jnp.int32),
        pltpu.VMEM((3 * NB2,), jnp.int32),
        pltpu.VMEM((3 * NB2,), jnp.int32),
        pltpu.VMEM((16,), jnp.int32),
        pltpu.SemaphoreType.DMA,
        pltpu.SemaphoreType.DMA,
    ],
)
def _k3_hist3(lat_hbm, hist2_hbm, thr1_hbm, nk_hbm, hist3_hbm, thr2_hbm,
              chunk0_v, chunk1_v, hist_v, red2_v, wchunk_v, thr_v,
              sem0, sem1):
    wid = _worker_id()
    lane = _lane()
    _reduce_workers(hist2_hbm, 3 * NB2, wchunk_v, red2_v)
    pltpu.sync_copy(thr1_hbm, thr_v)
    t1 = thr_v[...]
    pltpu.sync_copy(nk_hbm, thr_v)
    nk = jnp.sum(jnp.where(lane == 0, thr_v[...], 0))

    T21 = []
    CA2 = []
    for dd in range(3):
        b1 = _sel(t1, dd)
        ca1 = _sel(t1, 3 + dd)

        def load_cnt(vi, _dd=dd):
            return red2_v[pl.ds(_dd * NB2 + vi * 16, 16)]
        b2, ca2 = _find_split(load_cnt, NB2, ca1, nk)
        T21.append(b1 * NB2 + b2)
        CA2.append(ca2)

    _zero_i32(hist_v, 3 * NB3 * 16)
    ones = jnp.ones((16,), jnp.int32)
    T21u = [lax.convert_element_type(t, jnp.uint32) for t in T21]

    def vreg_fn(c, rl, i, v, cy):
        key = _key_of(v)
        p21 = key >> jnp.uint32(11)
        b3 = lax.convert_element_type(key & jnp.uint32(NB3 - 1), jnp.int32)
        for dd in range(GRP[c], 3):
            mask = p21 == T21u[dd]
            plsc.addupdate_scatter(
                hist_v, [dd * (NB3 * 16) + b3 * 16 + lane], ones, mask=mask)
        return cy

    _scan_latents(lat_hbm, (chunk0_v, chunk1_v), (sem0, sem1),
                  wid, vreg_fn, jnp.int32(0))
    _lane_reduce(hist_v, red2_v, 3 * NB3)
    pltpu.sync_copy(red2_v.at[pl.ds(0, 3 * NB3)],
                    hist3_hbm.at[pl.ds(wid * 3 * NB3, 3 * NB3)])

    thr = jnp.zeros((16,), jnp.int32)
    for dd in range(3):
        thr = jnp.where(lane == dd, T21[dd], thr)
        thr = jnp.where(lane == 3 + dd, CA2[dd], thr)
    thr_v[...] = thr

    @pl.when(wid == 0)
    def _():
        pltpu.sync_copy(thr_v, thr2_hbm)


@functools.partial(
    pl.kernel, mesh=_MESH,
    compiler_params=pltpu.CompilerParams(needs_layout_passes=False),
    out_type=(jax.ShapeDtypeStruct((N_TOKENS * DICT_SIZES[0],), jnp.float32),
              jax.ShapeDtypeStruct((N_TOKENS * DICT_SIZES[1],), jnp.float32),
              jax.ShapeDtypeStruct((N_TOKENS * DICT_SIZES[2],), jnp.float32)),
    scratch_types=(
        [pltpu.VMEM((CH,), jnp.float32) for _ in range(7)]
        + [
            pltpu.VMEM((3 * NB3,), jnp.int32),
            pltpu.VMEM((3 * NB3,), jnp.int32),
            pltpu.VMEM((16,), jnp.int32),
            pltpu.VMEM((32,), jnp.int32),
            pltpu.VMEM((32,), jnp.int32),
        ]
        + [pltpu.SemaphoreType.DMA for _ in range(8)]
    ),
)
def _k4_select(lat_hbm, hist3_hbm, thr2_hbm, nk_hbm,
               sp1_hbm, sp2_hbm, sp3_hbm,
               chunk0_v, chunk1_v, ob00_v, ob10_v, ob11_v, ob20_v, ob21_v,
               red3_v, wchunk_v, thr_v, gidx_v, band_v,
               sem, semi0, semi1, so00, so10, so11, so20, so21):
    wid = _worker_id()
    lane = _lane()
    _reduce_workers(hist3_hbm, 3 * NB3, wchunk_v, red3_v)
    pltpu.sync_copy(thr2_hbm, thr_v)
    t2 = thr_v[...]
    pltpu.sync_copy(nk_hbm, thr_v)
    nk = jnp.sum(jnp.where(lane == 0, thr_v[...], 0))

    T32 = []
    Q = []
    for dd in range(3):
        t21 = _sel(t2, dd)
        ca2 = _sel(t2, 3 + dd)

        def load_cnt(vi, _dd=dd):
            return red3_v[pl.ds(_dd * NB3 + vi * 16, 16)]
        b3, ca3 = _find_split(load_cnt, NB3, ca2, nk)
        T32.append((lax.convert_element_type(t21, jnp.uint32)
                    << jnp.uint32(11))
                   | lax.convert_element_type(b3, jnp.uint32))
        need = nk - ca3
        off = dd * NB3 + b3
        gidx_v[pl.ds(0, 16)] = lane * (3 * NB3) + off
        gidx_v[pl.ds(16, 16)] = (lane + 16) * (3 * NB3) + off
        pltpu.async_copy(hist3_hbm.at[gidx_v], band_v, sem).wait()
        c0 = band_v[pl.ds(0, 16)]
        c1 = band_v[pl.ds(16, 16)]
        ex0 = plsc.cumsum(c0) - c0
        ex1 = plsc.cumsum(c1) - c1 + jnp.sum(c0)
        pref = jnp.where(wid < 16, _sel(ex0, wid), _sel(ex1, wid - 16))
        cnt = jnp.where(wid < 16, _sel(c0, wid), _sel(c1, wid - 16))
        quota = jnp.clip(need - pref, 0, cnt)
        Q.append(jnp.broadcast_to(quota, (16,)))

    OB = {0: {0: ob00_v}, 1: {0: ob10_v, 1: ob11_v}, 2: {0: ob20_v, 1: ob21_v}}
    SO = {0: {0: so00}, 1: {0: so10, 1: so11}, 2: {0: so20, 1: so21}}
    sps = (sp1_hbm, sp2_hbm, sp3_hbm)
    inbufs = (chunk0_v, chunk1_v)
    insems = (semi0, semi1)
    zero16 = jnp.zeros((16,), jnp.float32)

    def in_slice(rl, c):
        return lat_hbm.at[pl.ds((wid * RW + rl) * MAX_D + c * CH, CH)]

    def out_dma(dd, p, r, c):
        d = DICT_SIZES[dd]
        return pltpu.make_async_copy(
            OB[dd][p], sps[dd].at[pl.ds(r * d + c * CH, CH)], SO[dd][p])

    pltpu.make_async_copy(in_slice(0, 0), inbufs[0], insems[0]).start()

    def row_body(rl, cy):
        qs = list(cy)
        r = wid * RW + rl
        for c in range(NCH):
            p = c % 2
            pltpu.make_async_copy(in_slice(rl, c), inbufs[p], insems[p]).wait()
            if c < NCH - 1:
                pltpu.make_async_copy(
                    in_slice(rl, c + 1), inbufs[(c + 1) % 2],
                    insems[(c + 1) % 2]).start()
            else:
                @pl.when(rl + 1 < RW)
                def _():
                    pltpu.make_async_copy(
                        in_slice(rl + 1, 0), inbufs[0], insems[0]).start()
            dds = list(range(GRP[c], 3))
            for dd in dds:
                if c <= 1:
                    @pl.when(rl > 0)
                    def _(dd=dd):
                        out_dma(dd, p, r, c).wait()
                else:
                    out_dma(dd, p, r, c).wait()

            @plsc.parallel_loop(0, NVR, step=1, unroll=UNROLL,
                                carry=tuple(qs))
            def _body(i, vcy, _dds=tuple(dds), _p=p):
                qq = list(vcy)
                v = inbufs[_p][pl.ds(i * 16, 16)]
                key = _key_of(v)
                for dd in _dds:
                    sure = key > T32[dd]
                    band = key == T32[dd]
                    csum = plsc.cumsum(lax.convert_element_type(
                        band, jnp.int32))
                    fill = band & (csum <= qq[dd])
                    emit = sure | fill
                    OB[dd][_p][pl.ds(i * 16, 16)] = jnp.where(
                        emit, v, zero16)
                    qq[dd] = qq[dd] - plsc.all_reduce_population_count(fill)
                return tuple(qq)

            qs = list(_body)
            for dd in dds:
                out_dma(dd, p, r, c).start()
        return tuple(qs)

    lax.fori_loop(0, RW, row_body, tuple(Q))
    last_r = N_TOKENS - 1
    for dd, p, c_last in ((0, 0, 0), (1, 0, 2), (1, 1, 3), (2, 0, 6), (2, 1, 7)):
        out_dma(dd, p, last_r, c_last).wait()


def kernel(x, enc_w, enc_b, bias, k):
    b = x.shape[0]
    latents = _encode(x, enc_w, bias, enc_b)
    lat_flat = latents.reshape(-1)
    nk = jnp.full((16,), jnp.minimum(k * b, KB), dtype=jnp.int32)

    hist1 = _k1_hist1(lat_flat)
    hist2, thr1 = _k2_hist2(lat_flat, hist1, nk)
    hist3, thr2 = _k3_hist3(lat_flat, hist2, thr1, nk)
    sp1, sp2, sp3 = _k4_select(lat_flat, hist3, thr2, nk)

    recons = []
    for dd, sp in enumerate((sp1, sp2, sp3)):
        d = DICT_SIZES[dd]
        sparse = sp.reshape(b, d)
        recons.append(_decode(sparse, enc_w[:d, :], bias))
    return tuple(recons)
